# baseline (speedup 1.0000x reference)
import functools

import jax
import jax.numpy as jnp
from jax import lax
from jax.experimental import pallas as pl
from jax.experimental.pallas import tpu as pltpu
from jax.experimental.pallas import tpu_sc as plsc

DIM = 64
B = 4096
FIELDS = 26
TOTAL = B * FIELDS
BPC = 4
CHUNK = BPC * FIELDS
NC = 2
NS = 16
NW = NC * NS
IPW = TOTAL // NW
CPW = IPW // CHUNK
NBUF = 6
GLAG = 3

_mesh = plsc.VectorSubcoreMesh(core_axis_name="c", subcore_axis_name="s")


@functools.partial(
    pl.kernel,
    mesh=_mesh,
    out_type=jax.ShapeDtypeStruct((B, 32, 128), jnp.float32),
    scratch_types=(
        [pltpu.VMEM((IPW,), jnp.int32)]
        + [pltpu.VMEM((CHUNK, DIM), jnp.float32) for _ in range(NBUF)]
        + [pltpu.SemaphoreType.DMA for _ in range(2 * NBUF)]
    ),
    compiler_params=pltpu.CompilerParams(use_tc_tiling_on_sc=False),
)
def _sc_gather(x_hbm, w_hbm, out_hbm, idx_v, *rest):
    bufs = rest[:NBUF]
    sg = rest[NBUF:2 * NBUF]
    ss = rest[2 * NBUF:3 * NBUF]
    wid = lax.axis_index("s") * NC + lax.axis_index("c")
    xoff = pl.multiple_of(wid * IPW, CHUNK)
    pltpu.sync_copy(x_hbm.at[pl.ds(xoff, IPW)], idx_v)
    bbase = pl.multiple_of(wid * (B // NW), BPC)

    gd = [None] * CPW
    sd = [[None] * BPC for _ in range(CPW)]

    def start_gather(j):
        b = j % NBUF
        gd[j] = pltpu.async_copy(
            w_hbm.at[idx_v.at[pl.ds(j * CHUNK, CHUNK)]], bufs[b], sg[b])

    def start_store(j):
        b = j % NBUF
        for k in range(BPC):
            sd[j][k] = pltpu.async_copy(
                bufs[b].at[pl.ds(k * FIELDS, FIELDS)],
                out_hbm.at[bbase + j * BPC + k, pl.ds(0, FIELDS), pl.ds(0, DIM)],
                ss[b])

    def wait_store(j):
        for k in range(BPC):
            sd[j][k].wait()

    for j in range(GLAG):
        start_gather(j)
    for j in range(CPW):
        gd[j].wait()
        start_store(j)
        nj = j + GLAG
        if nj < CPW:
            pj = nj - NBUF
            if pj >= 0:
                wait_store(pj)
            start_gather(nj)
    for j in range(CPW - NBUF, CPW):
        wait_store(j)


NUMROWS = 100000
_TK = 512
_TGRID = -(-NUMROWS // _TK)


@functools.partial(
    pl.pallas_call,
    grid=(_TGRID,),
    in_specs=[pl.BlockSpec((DIM, _TK), lambda i: (0, i))],
    out_specs=pl.BlockSpec((_TK // 2, 2 * DIM), lambda i: (i, 0)),
    out_shape=jax.ShapeDtypeStruct((NUMROWS // 2, 2 * DIM), jnp.float32),
)
def _tc_repack(wt_ref, out_ref):
    wt = wt_ref[...]
    ie = jnp.broadcast_to(jnp.arange(0, 128, 2)[None, :], (DIM, 64))
    io = jnp.broadcast_to(jnp.arange(1, 128, 2)[None, :], (DIM, 64))
    evs, ods = [], []
    for c in range(_TK // 128):
        chunk = lax.slice(wt, (0, c * 128), (DIM, (c + 1) * 128))
        evs.append(jnp.take_along_axis(chunk, ie, axis=1))
        ods.append(jnp.take_along_axis(chunk, io, axis=1))
    ev = jnp.concatenate(evs, axis=1)
    od = jnp.concatenate(ods, axis=1)
    out_ref[...] = jnp.concatenate([ev.T, od.T], axis=1)


def kernel(x, weight):
    wlin = _tc_repack(weight.T).reshape(NUMROWS, DIM)
    padded = _sc_gather(x.reshape(TOTAL), wlin)
    return padded[:, :FIELDS, :DIM]


# speedup vs baseline: 1.5938x; 1.5938x over previous
import functools

import jax
import jax.numpy as jnp
from jax import lax
from jax.experimental import pallas as pl
from jax.experimental.pallas import tpu as pltpu
from jax.experimental.pallas import tpu_sc as plsc

DIM = 64
B = 4096
FIELDS = 26
TOTAL = B * FIELDS
BPC = 4
CHUNK = BPC * FIELDS
NC = 2
NS = 16
NW = NC * NS
IPW = TOTAL // NW
CPW = IPW // CHUNK
NBUF = 6
GLAG = 3

_mesh = plsc.VectorSubcoreMesh(core_axis_name="c", subcore_axis_name="s")


@functools.partial(
    pl.kernel,
    mesh=_mesh,
    out_type=jax.ShapeDtypeStruct((B, 32, 128), jnp.float32),
    scratch_types=(
        [pltpu.VMEM((IPW,), jnp.int32)]
        + [pltpu.VMEM((CHUNK, 2 * DIM), jnp.float32) for _ in range(NBUF)]
        + [pltpu.SemaphoreType.DMA for _ in range(2 * NBUF)]
    ),
    compiler_params=pltpu.CompilerParams(use_tc_tiling_on_sc=False),
)
def _sc_gather(x_hbm, w_hbm, out_hbm, idx_v, *rest):
    bufs = rest[:NBUF]
    sg = rest[NBUF:2 * NBUF]
    ss = rest[2 * NBUF:3 * NBUF]
    wid = lax.axis_index("s") * NC + lax.axis_index("c")
    xoff = pl.multiple_of(wid * IPW, CHUNK)
    pltpu.sync_copy(x_hbm.at[pl.ds(xoff, IPW)], idx_v)
    bbase = pl.multiple_of(wid * (B // NW), BPC)

    gd = [None] * CPW
    sd = [[None] * BPC for _ in range(CPW)]

    def start_gather(j):
        b = j % NBUF
        gd[j] = pltpu.async_copy(
            w_hbm.at[idx_v.at[pl.ds(j * CHUNK, CHUNK)]], bufs[b], sg[b])

    def start_store(j):
        b = j % NBUF
        for k in range(BPC):
            sd[j][k] = pltpu.async_copy(
                bufs[b].at[pl.ds(k * FIELDS, FIELDS), pl.ds(0, DIM)],
                out_hbm.at[bbase + j * BPC + k, pl.ds(0, FIELDS), pl.ds(0, DIM)],
                ss[b])

    def wait_store(j):
        for k in range(BPC):
            sd[j][k].wait()

    for j in range(GLAG):
        start_gather(j)
    for j in range(CPW):
        gd[j].wait()
        start_store(j)
        nj = j + GLAG
        if nj < CPW:
            pj = nj - NBUF
            if pj >= 0:
                wait_store(pj)
            start_gather(nj)
    for j in range(CPW - NBUF, CPW):
        wait_store(j)


def kernel(x, weight):
    wpad = jnp.pad(weight, ((0, 0), (0, DIM)))
    padded = _sc_gather(x.reshape(TOTAL), wpad)
    return padded[:, :FIELDS, :DIM]


# speedup vs baseline: 1.6132x; 1.0122x over previous
import functools

import jax
import jax.numpy as jnp
from jax import lax
from jax.experimental import pallas as pl
from jax.experimental.pallas import tpu as pltpu
from jax.experimental.pallas import tpu_sc as plsc

DIM = 64
B = 4096
FIELDS = 26
TOTAL = B * FIELDS
BPC = 4
CHUNK = BPC * FIELDS
NC = 2
NS = 16
NW = NC * NS
IPW = TOTAL // NW
CPW = IPW // CHUNK
NBUF = 8
GLAG = 4

_mesh = plsc.VectorSubcoreMesh(core_axis_name="c", subcore_axis_name="s")


@functools.partial(
    pl.kernel,
    mesh=_mesh,
    out_type=jax.ShapeDtypeStruct((B, 32, 128), jnp.float32),
    scratch_types=(
        [pltpu.VMEM((IPW,), jnp.int32)]
        + [pltpu.VMEM((CHUNK, DIM), jnp.float32) for _ in range(NBUF)]
        + [pltpu.SemaphoreType.DMA for _ in range(2 * NBUF)]
    ),
    compiler_params=pltpu.CompilerParams(use_tc_tiling_on_sc=False),
)
def _sc_gather(x_hbm, w_hbm, out_hbm, idx_v, *rest):
    bufs = rest[:NBUF]
    sg = rest[NBUF:2 * NBUF]
    ss = rest[2 * NBUF:3 * NBUF]
    wid = lax.axis_index("s") * NC + lax.axis_index("c")
    xoff = pl.multiple_of(wid * IPW, CHUNK)
    pltpu.sync_copy(x_hbm.at[pl.ds(xoff, IPW)], idx_v)
    bbase = pl.multiple_of(wid * (B // NW), BPC)

    gd = [None] * CPW
    sd = [[None] * BPC for _ in range(CPW)]

    def start_gather(j):
        b = j % NBUF
        gd[j] = pltpu.async_copy(
            w_hbm.at[idx_v.at[pl.ds(j * CHUNK, CHUNK)]], bufs[b], sg[b])

    def start_store(j):
        b = j % NBUF
        for k in range(BPC):
            sd[j][k] = pltpu.async_copy(
                bufs[b].at[pl.ds(k * FIELDS, FIELDS)],
                out_hbm.at[bbase + j * BPC + k, pl.ds(0, FIELDS), pl.ds(0, DIM)],
                ss[b])

    def wait_store(j):
        for k in range(BPC):
            sd[j][k].wait()

    for j in range(GLAG):
        start_gather(j)
    for j in range(CPW):
        gd[j].wait()
        start_store(j)
        nj = j + GLAG
        if nj < CPW:
            pj = nj - NBUF
            if pj >= 0:
                wait_store(pj)
            start_gather(nj)
    for j in range(CPW - NBUF, CPW):
        wait_store(j)


def kernel(x, weight):
    padded = _sc_gather(x.reshape(TOTAL), weight)
    return padded[:, :FIELDS, :DIM]
